# baseline (speedup 1.0000x reference)
import math

import jax
import jax.numpy as jnp
from jax.experimental import pallas as pl

HID = 64
GROUP = 4
OUT = 256
NSRC = 10
_TAB_OF_SRC = [0, 1, 2, 3, 4, 5, None, None, 6, 7]
NROW = 500
NG = 50


def _prologue_body(tabs_ref, dxW1_ref, dyW1_ref, dxb1_ref, dyb1_ref,
                   dxW2_ref, dyW2_ref, dxb2_ref, dyb2_ref,
                   ffnW_ref, ffnb_ref,
                   C_ref, Dbig_ref, KX_ref, KY_ref, B1_ref, MX_ref, MY_ref):
    W = ffnW_ref[...]
    T0 = tabs_ref[:, 0, :]
    DT = tabs_ref[:, 1, :] - T0
    T0t = jnp.concatenate([T0] * GROUP, axis=1)
    b2x = jnp.concatenate([dxb2_ref[...]] * GROUP, axis=1)
    b2y = jnp.concatenate([dyb2_ref[...]] * GROUP, axis=1)
    SRC = jnp.concatenate([T0t[0:6], b2x, b2y, T0t[6:8]], axis=0)
    BASE10 = jnp.dot(SRC, W, preferred_element_type=jnp.float32)

    row = jax.lax.broadcasted_iota(jnp.int32, (NROW, OUT), 0).astype(jnp.float32)
    col = jax.lax.broadcasted_iota(jnp.int32, (NROW, OUT), 1)
    j = col // HID
    d = col % HID
    p = row * float(GROUP) + j.astype(jnp.float32)
    dd = ((d // 2) * 2).astype(jnp.float32)
    freq = jnp.exp(dd * (-math.log(10000.0) / HID))
    ang = p * freq
    pe_r = jnp.where(d % 2 == 0, jnp.sin(ang), jnp.cos(ang))

    C = jnp.dot(pe_r, W, preferred_element_type=jnp.float32) + ffnb_ref[...]
    C = C + jnp.broadcast_to(BASE10[:, None, :], (NSRC, NG, OUT)
                             ).reshape(NROW, OUT)
    C_ref[...] = C

    DJ = []
    for jj in range(GROUP):
        Wj = W[HID * jj:HID * (jj + 1), :]
        DJ.append(jnp.dot(DT, Wj, preferred_element_type=jnp.float32))
    zrow = jnp.zeros((1, OUT), jnp.float32)
    rows = []
    for s10 in range(NSRC):
        m = _TAB_OF_SRC[s10]
        for jj in range(GROUP):
            rows.append(zrow if m is None else DJ[jj][m:m + 1])
    Dbig_ref[...] = jnp.concatenate(rows, axis=0)

    zero = jnp.zeros((1, HID), jnp.float32)
    rowsx, rowsy = [], []
    for jj in range(GROUP):
        px = [dxW1_ref[...] if k == jj else zero for k in range(GROUP)]
        py = [dyW1_ref[...] if k == jj else zero for k in range(GROUP)]
        rowsx.append(jnp.concatenate(px, axis=1))
        rowsy.append(jnp.concatenate(py, axis=1))
    KX_ref[...] = jnp.concatenate(rowsx, axis=0)
    KY_ref[...] = jnp.concatenate(rowsy, axis=0)
    B1_ref[...] = jnp.concatenate(
        [jnp.concatenate([dxb1_ref[...]] * GROUP, axis=1),
         jnp.concatenate([dyb1_ref[...]] * GROUP, axis=1)], axis=0)

    mx, my = [], []
    for jj in range(GROUP):
        Wj = W[HID * jj:HID * (jj + 1), :]
        mx.append(jnp.dot(dxW2_ref[...], Wj, preferred_element_type=jnp.float32))
        my.append(jnp.dot(dyW2_ref[...], Wj, preferred_element_type=jnp.float32))
    MX_ref[...] = jnp.concatenate(mx, axis=0)
    MY_ref[...] = jnp.concatenate(my, axis=0)


def _main_body(X_ref, C_ref, Dbig_ref, KX_ref, KY_ref, B1_ref, MX_ref, MY_ref,
               lnw_ref, out_ref):
    Bt = out_ref.shape[0]
    lnw = lnw_ref[...]
    C2 = C_ref[...]
    Dbig = Dbig_ref[...]

    src_of_row = jax.lax.broadcasted_iota(jnp.int32, (NROW, GROUP), 0) // NG
    Xall = X_ref[...]
    pieces = []
    for m in range(NSRC):
        mask = (src_of_row == m).astype(jnp.float32)[None]
        pieces.append(Xall * mask)
    Xbig = jnp.concatenate(pieces, axis=2)

    for b in range(Bt):
        Xb = Xbig[b]
        acc = jnp.dot(Xb, Dbig, preferred_element_type=jnp.float32) + C2
        adds = []
        for lo, cb, K1_ref, M_ref, ib in ((6 * NG, 24, KX_ref, MX_ref, 0),
                                          (7 * NG, 28, KY_ref, MY_ref, 1)):
            feats = Xb[lo:lo + NG, cb:cb + GROUP]
            pre = jnp.dot(feats, K1_ref[...], preferred_element_type=jnp.float32)
            h = jnp.maximum(pre + B1_ref[ib][None], 0.0)
            adds.append(jnp.dot(h, M_ref[...], preferred_element_type=jnp.float32))
        acc = jnp.concatenate(
            [acc[:6 * NG], acc[6 * NG:7 * NG] + adds[0],
             acc[7 * NG:8 * NG] + adds[1], acc[8 * NG:]], axis=0)
        ms = jnp.mean(acc * acc, axis=-1, keepdims=True)
        out_ref[b] = acc * jax.lax.rsqrt(ms + 1e-6) * lnw


def kernel(w, a, s, d, space, shift, mouse_1, mouse_2, dx, dy, w_tab, a_tab,
           s_tab, d_tab, space_tab, shift_tab, m1_tab, m2_tab, dx_W1, dx_b1,
           dx_W2, dx_b2, dy_W1, dy_b1, dy_W2, dy_b2, ffn_W, ffn_b, ln_w):
    B, T = w.shape
    f32 = jnp.float32

    tabs = jnp.stack([w_tab, a_tab, s_tab, d_tab, space_tab, shift_tab,
                      m1_tab, m2_tab])
    X = jnp.concatenate(
        [w.astype(f32), a.astype(f32), s.astype(f32), d.astype(f32),
         space.astype(f32), shift.astype(f32), dx, dy,
         mouse_1.astype(f32), mouse_2.astype(f32)], axis=1)
    X = X.reshape(B, NROW, GROUP)

    C, Dbig, KX, KY, B1, MX, MY = pl.pallas_call(
        _prologue_body,
        out_shape=[
            jax.ShapeDtypeStruct((NROW, OUT), f32),
            jax.ShapeDtypeStruct((NSRC * GROUP, OUT), f32),
            jax.ShapeDtypeStruct((GROUP, OUT), f32),
            jax.ShapeDtypeStruct((GROUP, OUT), f32),
            jax.ShapeDtypeStruct((2, OUT), f32),
            jax.ShapeDtypeStruct((OUT, OUT), f32),
            jax.ShapeDtypeStruct((OUT, OUT), f32),
        ],
    )(tabs, dx_W1, dy_W1, dx_b1.reshape(1, HID), dy_b1.reshape(1, HID),
      dx_W2, dy_W2, dx_b2.reshape(1, HID), dy_b2.reshape(1, HID),
      ffn_W, ffn_b.reshape(1, OUT))

    Bt = 16
    grid = (B // Bt,)
    full = lambda shape: pl.BlockSpec(shape, lambda i: (0,) * len(shape))
    out = pl.pallas_call(
        _main_body,
        grid=grid,
        in_specs=[
            pl.BlockSpec((Bt, NROW, GROUP), lambda i: (i, 0, 0)),
            full((NROW, OUT)), full((NSRC * GROUP, OUT)),
            full((GROUP, OUT)), full((GROUP, OUT)), full((2, OUT)),
            full((OUT, OUT)), full((OUT, OUT)),
            full((1, OUT)),
        ],
        out_specs=pl.BlockSpec((Bt, NROW, OUT), lambda i: (i, 0, 0)),
        out_shape=jax.ShapeDtypeStruct((B, NROW, OUT), f32),
    )(X, C, Dbig, KX, KY, B1, MX, MY, ln_w.reshape(1, OUT))
    return out


# speedup vs baseline: 1.0656x; 1.0656x over previous
import functools
import math

import jax
import jax.numpy as jnp
from jax import lax
from jax.experimental import pallas as pl
from jax.experimental.pallas import tpu as pltpu
from jax.experimental.pallas import tpu_sc as plsc

HID = 64
GROUP = 4
OUT = 256
NSRC = 10
_TAB_OF_SRC = [0, 1, 2, 3, 4, 5, None, None, 6, 7]
NROW = 500
NG = 50
NNIB = 16
TPAD = 9600


def _prologue_body(tabs_ref, dxW1_ref, dyW1_ref, dxb1_ref, dyb1_ref,
                   dxW2_ref, dyW2_ref, dxb2_ref, dyb2_ref,
                   ffnW_ref, ffnb_ref, lnw_ref,
                   TBL_ref, CD_ref, KX_ref, KY_ref, B1_ref, MX_ref, MY_ref):
    W = ffnW_ref[...]
    T0 = tabs_ref[:, 0, :]
    DT = tabs_ref[:, 1, :] - T0
    T0t = jnp.concatenate([T0] * GROUP, axis=1)
    b2x = jnp.concatenate([dxb2_ref[...]] * GROUP, axis=1)
    b2y = jnp.concatenate([dyb2_ref[...]] * GROUP, axis=1)
    SRC = jnp.concatenate([T0t[0:6], b2x, b2y, T0t[6:8]], axis=0)
    BASE10 = jnp.dot(SRC, W, preferred_element_type=jnp.float32)

    row = jax.lax.broadcasted_iota(jnp.int32, (NROW, OUT), 0).astype(jnp.float32)
    col = jax.lax.broadcasted_iota(jnp.int32, (NROW, OUT), 1)
    j = col // HID
    d = col % HID
    p = row * float(GROUP) + j.astype(jnp.float32)
    dd = ((d // 2) * 2).astype(jnp.float32)
    freq = jnp.exp(dd * (-math.log(10000.0) / HID))
    ang = p * freq
    pe_r = jnp.where(d % 2 == 0, jnp.sin(ang), jnp.cos(ang))

    C = jnp.dot(pe_r, W, preferred_element_type=jnp.float32) + ffnb_ref[...]
    C = C + jnp.broadcast_to(BASE10[:, None, :], (NSRC, NG, OUT)
                             ).reshape(NROW, OUT)
    CD_ref[...] = C[6 * NG:8 * NG]

    DJ = []
    for jj in range(GROUP):
        Wj = W[HID * jj:HID * (jj + 1), :]
        DJ.append(jnp.dot(DT, Wj, preferred_element_type=jnp.float32))

    ni = jax.lax.broadcasted_iota(jnp.int32, (NNIB, GROUP), 0)
    ji = jax.lax.broadcasted_iota(jnp.int32, (NNIB, GROUP), 1)
    NB = ((ni >> ji) & 1).astype(jnp.float32)

    lnw = lnw_ref[...]
    for s10 in range(NSRC):
        m = _TAB_OF_SRC[s10]
        if m is None:
            contrib = jnp.zeros((NNIB, OUT), jnp.float32)
        else:
            Ds = jnp.concatenate([DJ[jj][m:m + 1] for jj in range(GROUP)],
                                 axis=0)
            contrib = jnp.dot(NB, Ds, preferred_element_type=jnp.float32)
        pre = C[NG * s10:NG * (s10 + 1)][:, None, :] + contrib[None]
        ms = jnp.mean(pre * pre, axis=-1, keepdims=True)
        nrm = (pre * jax.lax.rsqrt(ms + 1e-6) * lnw[None]).reshape(
            NG * NNIB, OUT)
        TBL_ref[pl.ds(s10 * NG * NNIB, NG * NNIB), :] = nrm
    TBL_ref[pl.ds(NROW * NNIB, TPAD - NROW * NNIB), :] = jnp.zeros(
        (TPAD - NROW * NNIB, OUT), jnp.float32)

    zero = jnp.zeros((1, HID), jnp.float32)
    rowsx, rowsy = [], []
    for jj in range(GROUP):
        px = [dxW1_ref[...] if k == jj else zero for k in range(GROUP)]
        py = [dyW1_ref[...] if k == jj else zero for k in range(GROUP)]
        rowsx.append(jnp.concatenate(px, axis=1))
        rowsy.append(jnp.concatenate(py, axis=1))
    KX_ref[...] = jnp.concatenate(rowsx, axis=0)
    KY_ref[...] = jnp.concatenate(rowsy, axis=0)
    B1_ref[...] = jnp.concatenate(
        [jnp.concatenate([dxb1_ref[...]] * GROUP, axis=1),
         jnp.concatenate([dyb1_ref[...]] * GROUP, axis=1)], axis=0)

    mx, my = [], []
    for jj in range(GROUP):
        Wj = W[HID * jj:HID * (jj + 1), :]
        mx.append(jnp.dot(dxW2_ref[...], Wj, preferred_element_type=jnp.float32))
        my.append(jnp.dot(dyW2_ref[...], Wj, preferred_element_type=jnp.float32))
    MX_ref[...] = jnp.concatenate(mx, axis=0)
    MY_ref[...] = jnp.concatenate(my, axis=0)


_CHUNKS = [(0, 96), (96, 96), (192, 96), (288, 96), (384, 96), (480, 16)]
_NTAIL = 4


def _prep_body(X_ref, CD_ref, KX_ref, KY_ref, B1_ref, MX_ref, MY_ref,
               lnw_ref, idx_ref, DXY_ref):
    Bt = X_ref.shape[0]
    lnw = lnw_ref[...]

    x0 = X_ref[:, :, 0]
    x1 = X_ref[:, :, 1]
    x2 = X_ref[:, :, 2]
    x3 = X_ref[:, :, 3]
    nib = x0 + 2.0 * x1 + 4.0 * x2 + 8.0 * x3
    g = jax.lax.broadcasted_iota(jnp.int32, (1, NROW), 1).astype(jnp.float32)
    idx = (nib + float(NNIB) * g).astype(jnp.int32)
    bcol = (pl.program_id(0) * Bt
            + jax.lax.broadcasted_iota(jnp.int32, (Bt, 1), 0))
    gi = jax.lax.broadcasted_iota(jnp.int32, (1, NROW), 1)
    dxyrow = TPAD + bcol * (2 * NG) + (gi - 6 * NG)
    in_dxy = jnp.logical_and(gi >= 6 * NG, gi < 8 * NG)
    idx = jnp.where(in_dxy, dxyrow, idx)

    last = idx[:, NROW - 1:NROW]
    idxp = jnp.concatenate([idx] + [last] * (512 - NROW), axis=1)
    rows = []
    for o, n in _CHUNKS:
        take = min(128, 512 - o)
        r = idxp[:, o:o + take]
        if take < 128:
            r = jnp.concatenate([r, jnp.zeros((Bt, 128 - take), jnp.int32)],
                                axis=1)
        rows.append(r)
    tail = jnp.concatenate(
        [idxp[:, NROW - _NTAIL:NROW],
         jnp.zeros((Bt, 124), jnp.int32)], axis=1)
    rows.append(tail)
    rows.append(jnp.zeros((Bt, 128), jnp.int32))
    idx_ref[...] = jnp.stack(rows, axis=1)

    XD = X_ref[:, 6 * NG:8 * NG, :]
    parts = []
    for half, (K1_ref, M_ref, ib) in enumerate(((KX_ref, MX_ref, 0),
                                                (KY_ref, MY_ref, 1))):
        feats = XD[:, half * NG:(half + 1) * NG, :].reshape(Bt * NG, GROUP)
        pre = jnp.dot(feats, K1_ref[...], preferred_element_type=jnp.float32)
        h = jnp.maximum(pre + B1_ref[ib][None], 0.0)
        parts.append(jnp.dot(h, M_ref[...], preferred_element_type=jnp.float32)
                     .reshape(Bt, NG, OUT))
    acc = jnp.concatenate(parts, axis=1) + CD_ref[...][None]
    ms = jnp.mean(acc * acc, axis=-1, keepdims=True)
    DXY_ref[...] = (acc * jax.lax.rsqrt(ms + 1e-6) * lnw[None]).reshape(
        Bt * 2 * NG, OUT)


def _make_sc_gather(B, nsub):
    nb = B // nsub
    mesh = plsc.VectorSubcoreMesh(core_axis_name="c", subcore_axis_name="s")

    @functools.partial(
        pl.kernel, mesh=mesh,
        out_type=jax.ShapeDtypeStruct((B, NROW, OUT), jnp.float32),
        scratch_types=[
            pltpu.VMEM((8, 128), jnp.int32),
            pltpu.VMEM((NROW, OUT), jnp.float32),
            pltpu.VMEM((4, OUT), jnp.float32),
            pltpu.SemaphoreType.DMA,
        ],
    )
    def sc_gather(srcall, idxh, out, idx_v, slab, tbuf, semg):
        wid = lax.axis_index("s") * 2 + lax.axis_index("c")

        def body(i, carry):
            b = wid * nb + i
            pltpu.sync_copy(idxh.at[b], idx_v)
            gets = [
                pltpu.async_copy(srcall.at[idx_v.at[c, pl.ds(0, n)]],
                                 slab.at[pl.ds(o, n)], semg)
                for c, (o, n) in enumerate(_CHUNKS)
            ]
            gets.append(
                pltpu.async_copy(srcall.at[idx_v.at[6, pl.ds(0, 4)]],
                                 tbuf, semg))
            for dd in gets:
                dd.wait()
            for r in range(_NTAIL):
                for k in range(OUT // 16):
                    slab[NROW - _NTAIL + r, pl.ds(16 * k, 16)] = (
                        tbuf[r, pl.ds(16 * k, 16)])
            pltpu.sync_copy(slab, out.at[b])
            return carry

        lax.fori_loop(0, nb, body, 0)

    return sc_gather


def kernel(w, a, s, d, space, shift, mouse_1, mouse_2, dx, dy, w_tab, a_tab,
           s_tab, d_tab, space_tab, shift_tab, m1_tab, m2_tab, dx_W1, dx_b1,
           dx_W2, dx_b2, dy_W1, dy_b1, dy_W2, dy_b2, ffn_W, ffn_b, ln_w):
    B, T = w.shape
    f32 = jnp.float32

    tabs = jnp.stack([w_tab, a_tab, s_tab, d_tab, space_tab, shift_tab,
                      m1_tab, m2_tab])
    X = jnp.concatenate(
        [w.astype(f32), a.astype(f32), s.astype(f32), d.astype(f32),
         space.astype(f32), shift.astype(f32), dx, dy,
         mouse_1.astype(f32), mouse_2.astype(f32)], axis=1)
    X = X.reshape(B, NROW, GROUP)
    lnw2 = ln_w.reshape(1, OUT)

    TBL, CD, KX, KY, B1, MX, MY = pl.pallas_call(
        _prologue_body,
        out_shape=[
            jax.ShapeDtypeStruct((TPAD, OUT), f32),
            jax.ShapeDtypeStruct((2 * NG, OUT), f32),
            jax.ShapeDtypeStruct((GROUP, OUT), f32),
            jax.ShapeDtypeStruct((GROUP, OUT), f32),
            jax.ShapeDtypeStruct((2, OUT), f32),
            jax.ShapeDtypeStruct((OUT, OUT), f32),
            jax.ShapeDtypeStruct((OUT, OUT), f32),
        ],
    )(tabs, dx_W1, dy_W1, dx_b1.reshape(1, HID), dy_b1.reshape(1, HID),
      dx_W2, dy_W2, dx_b2.reshape(1, HID), dy_b2.reshape(1, HID),
      ffn_W, ffn_b.reshape(1, OUT), lnw2)

    Bt = 32
    full = lambda shape: pl.BlockSpec(shape, lambda i: (0,) * len(shape))
    idx, DXY = pl.pallas_call(
        _prep_body,
        grid=(B // Bt,),
        in_specs=[
            pl.BlockSpec((Bt, NROW, GROUP), lambda i: (i, 0, 0)),
            full((2 * NG, OUT)), full((GROUP, OUT)), full((GROUP, OUT)),
            full((2, OUT)), full((OUT, OUT)), full((OUT, OUT)),
            full((1, OUT)),
        ],
        out_specs=[
            pl.BlockSpec((Bt, 8, 128), lambda i: (i, 0, 0)),
            pl.BlockSpec((Bt * 2 * NG, OUT), lambda i: (i, 0)),
        ],
        out_shape=[
            jax.ShapeDtypeStruct((B, 8, 128), jnp.int32),
            jax.ShapeDtypeStruct((B * 2 * NG, OUT), f32),
        ],
    )(X, CD, KX, KY, B1, MX, MY, lnw2)

    SRCALL = jnp.concatenate([TBL, DXY], axis=0)

    info = plsc.get_sparse_core_info()
    nsub = info.num_cores * info.num_subcores
    out = _make_sc_gather(B, nsub)(SRCALL, idx)
    return out


# speedup vs baseline: 1.2938x; 1.2142x over previous
import functools
import math

import jax
import jax.numpy as jnp
from jax import lax
from jax.experimental import pallas as pl
from jax.experimental.pallas import tpu as pltpu
from jax.experimental.pallas import tpu_sc as plsc

HID = 64
GROUP = 4
OUT = 256
NSRC = 10
_TAB_OF_SRC = [0, 1, 2, 3, 4, 5, None, None, 6, 7]
NROW = 500
NG = 50
NNIB = 16
TPAD = 9600


def _prologue_body(tabs_ref, dxW1_ref, dyW1_ref, dxb1_ref, dyb1_ref,
                   dxW2_ref, dyW2_ref, dxb2_ref, dyb2_ref,
                   ffnW_ref, ffnb_ref, lnw_ref,
                   TBL_ref, CD_ref, KX_ref, KY_ref, B1_ref, MX_ref, MY_ref,
                   P_ref):
    W = ffnW_ref[...]
    T0 = tabs_ref[:, 0, :]
    DT = tabs_ref[:, 1, :] - T0
    T0t = jnp.concatenate([T0] * GROUP, axis=1)
    b2x = jnp.concatenate([dxb2_ref[...]] * GROUP, axis=1)
    b2y = jnp.concatenate([dyb2_ref[...]] * GROUP, axis=1)
    SRC = jnp.concatenate([T0t[0:6], b2x, b2y, T0t[6:8]], axis=0)
    BASE10 = jnp.dot(SRC, W, preferred_element_type=jnp.float32)

    row = jax.lax.broadcasted_iota(jnp.int32, (NROW, OUT), 0).astype(jnp.float32)
    col = jax.lax.broadcasted_iota(jnp.int32, (NROW, OUT), 1)
    j = col // HID
    d = col % HID
    p = row * float(GROUP) + j.astype(jnp.float32)
    dd = ((d // 2) * 2).astype(jnp.float32)
    freq = jnp.exp(dd * (-math.log(10000.0) / HID))
    ang = p * freq
    pe_r = jnp.where(d % 2 == 0, jnp.sin(ang), jnp.cos(ang))

    C = jnp.dot(pe_r, W, preferred_element_type=jnp.float32) + ffnb_ref[...]
    C = C + jnp.broadcast_to(BASE10[:, None, :], (NSRC, NG, OUT)
                             ).reshape(NROW, OUT)
    CD_ref[...] = C[6 * NG:8 * NG]

    DJ = []
    for jj in range(GROUP):
        Wj = W[HID * jj:HID * (jj + 1), :]
        DJ.append(jnp.dot(DT, Wj, preferred_element_type=jnp.float32))

    ni = jax.lax.broadcasted_iota(jnp.int32, (NNIB, GROUP), 0)
    ji = jax.lax.broadcasted_iota(jnp.int32, (NNIB, GROUP), 1)
    NB = ((ni >> ji) & 1).astype(jnp.float32)

    lnw = lnw_ref[...]
    for s10 in range(NSRC):
        m = _TAB_OF_SRC[s10]
        if m is None:
            contrib = jnp.zeros((NNIB, OUT), jnp.float32)
        else:
            Ds = jnp.concatenate([DJ[jj][m:m + 1] for jj in range(GROUP)],
                                 axis=0)
            contrib = jnp.dot(NB, Ds, preferred_element_type=jnp.float32)
        pre = C[NG * s10:NG * (s10 + 1)][:, None, :] + contrib[None]
        ms = jnp.mean(pre * pre, axis=-1, keepdims=True)
        nrm = (pre * jax.lax.rsqrt(ms + 1e-6) * lnw[None]).reshape(
            NG * NNIB, OUT)
        TBL_ref[pl.ds(s10 * NG * NNIB, NG * NNIB), :] = nrm
    TBL_ref[pl.ds(NROW * NNIB, TPAD - NROW * NNIB), :] = jnp.zeros(
        (TPAD - NROW * NNIB, OUT), jnp.float32)

    zero = jnp.zeros((1, HID), jnp.float32)
    rowsx, rowsy = [], []
    for jj in range(GROUP):
        px = [dxW1_ref[...] if k == jj else zero for k in range(GROUP)]
        py = [dyW1_ref[...] if k == jj else zero for k in range(GROUP)]
        rowsx.append(jnp.concatenate(px, axis=1))
        rowsy.append(jnp.concatenate(py, axis=1))
    KX_ref[...] = jnp.concatenate(rowsx, axis=0)
    KY_ref[...] = jnp.concatenate(rowsy, axis=0)
    B1_ref[...] = jnp.concatenate(
        [jnp.concatenate([dxb1_ref[...]] * GROUP, axis=1),
         jnp.concatenate([dyb1_ref[...]] * GROUP, axis=1)], axis=0)

    mx, my = [], []
    for jj in range(GROUP):
        Wj = W[HID * jj:HID * (jj + 1), :]
        mx.append(jnp.dot(dxW2_ref[...], Wj, preferred_element_type=jnp.float32))
        my.append(jnp.dot(dyW2_ref[...], Wj, preferred_element_type=jnp.float32))
    MX_ref[...] = jnp.concatenate(mx, axis=0)
    MY_ref[...] = jnp.concatenate(my, axis=0)

    ti = jax.lax.broadcasted_iota(jnp.int32, (GROUP * NROW, NROW), 0)
    gi2 = jax.lax.broadcasted_iota(jnp.int32, (GROUP * NROW, NROW), 1)
    P_ref[...] = jnp.where(ti // GROUP == gi2,
                           (1 << (ti % GROUP)), 0).astype(jnp.float32)


_CHUNKS = [(0, 96), (96, 96), (192, 96), (288, 96), (384, 96), (480, 16)]
_NTAIL = 4


def _prep_body(X2_ref, XD_ref, P_ref, CD_ref, KX_ref, KY_ref, B1_ref,
               MX_ref, MY_ref, lnw_ref, idx_ref, DXY_ref):
    Bt = X2_ref.shape[0]
    lnw = lnw_ref[...]

    nib = jnp.dot(X2_ref[...], P_ref[...],
                  preferred_element_type=jnp.float32)
    g = jax.lax.broadcasted_iota(jnp.int32, (1, NROW), 1).astype(jnp.float32)
    bcol = (pl.program_id(0) * Bt
            + jax.lax.broadcasted_iota(jnp.int32, (Bt, 1), 0)).astype(
                jnp.float32)
    gi = jax.lax.broadcasted_iota(jnp.int32, (1, NROW), 1)
    dxyrow = float(TPAD) + bcol * float(2 * NG) + (
        g - float(6 * NG))
    in_dxy = jnp.logical_and(gi >= 6 * NG, gi < 8 * NG)
    idx = jnp.where(in_dxy, dxyrow, nib + float(NNIB) * g).astype(jnp.int32)

    last = idx[:, NROW - 1:NROW]
    idxp = jnp.concatenate([idx] + [last] * (512 - NROW), axis=1)
    rows = []
    for o, n in _CHUNKS:
        take = min(128, 512 - o)
        r = idxp[:, o:o + take]
        if take < 128:
            r = jnp.concatenate([r, jnp.zeros((Bt, 128 - take), jnp.int32)],
                                axis=1)
        rows.append(r)
    tail = jnp.concatenate(
        [idxp[:, NROW - _NTAIL:NROW],
         jnp.zeros((Bt, 124), jnp.int32)], axis=1)
    rows.append(tail)
    rows.append(jnp.zeros((Bt, 128), jnp.int32))
    idx_ref[...] = jnp.stack(rows, axis=1)

    XD = XD_ref[...]
    parts = []
    for half, (K1_ref, M_ref, ib) in enumerate(((KX_ref, MX_ref, 0),
                                                (KY_ref, MY_ref, 1))):
        feats = XD[:, half * NG:(half + 1) * NG, :].reshape(Bt * NG, GROUP)
        pre = jnp.dot(feats, K1_ref[...], preferred_element_type=jnp.float32)
        h = jnp.maximum(pre + B1_ref[ib][None], 0.0)
        parts.append(jnp.dot(h, M_ref[...], preferred_element_type=jnp.float32)
                     .reshape(Bt, NG, OUT))
    acc = jnp.concatenate(parts, axis=1) + CD_ref[...][None]
    ms = jnp.mean(acc * acc, axis=-1, keepdims=True)
    DXY_ref[...] = (acc * jax.lax.rsqrt(ms + 1e-6) * lnw[None]).reshape(
        Bt * 2 * NG, OUT)


def _make_sc_gather(B, nsub):
    nb = B // nsub
    mesh = plsc.VectorSubcoreMesh(core_axis_name="c", subcore_axis_name="s")

    @functools.partial(
        pl.kernel, mesh=mesh,
        out_type=jax.ShapeDtypeStruct((B, NROW, OUT), jnp.float32),
        scratch_types=[
            pltpu.VMEM((8, 128), jnp.int32),
            pltpu.VMEM((NROW, OUT), jnp.float32),
            pltpu.VMEM((4, OUT), jnp.float32),
            pltpu.SemaphoreType.DMA,
        ],
    )
    def sc_gather(srcall, idxh, out, idx_v, slab, tbuf, semg):
        wid = lax.axis_index("s") * 2 + lax.axis_index("c")

        def body(i, carry):
            b = wid * nb + i
            pltpu.sync_copy(idxh.at[b], idx_v)
            gets = [
                pltpu.async_copy(srcall.at[idx_v.at[c, pl.ds(0, n)]],
                                 slab.at[pl.ds(o, n)], semg)
                for c, (o, n) in enumerate(_CHUNKS)
            ]
            gets.append(
                pltpu.async_copy(srcall.at[idx_v.at[6, pl.ds(0, 4)]],
                                 tbuf, semg))
            for dd in gets:
                dd.wait()
            for r in range(_NTAIL):
                for k in range(OUT // 16):
                    slab[NROW - _NTAIL + r, pl.ds(16 * k, 16)] = (
                        tbuf[r, pl.ds(16 * k, 16)])
            pltpu.sync_copy(slab, out.at[b])
            return carry

        lax.fori_loop(0, nb, body, 0)

    return sc_gather


def kernel(w, a, s, d, space, shift, mouse_1, mouse_2, dx, dy, w_tab, a_tab,
           s_tab, d_tab, space_tab, shift_tab, m1_tab, m2_tab, dx_W1, dx_b1,
           dx_W2, dx_b2, dy_W1, dy_b1, dy_W2, dy_b2, ffn_W, ffn_b, ln_w):
    B, T = w.shape
    f32 = jnp.float32

    tabs = jnp.stack([w_tab, a_tab, s_tab, d_tab, space_tab, shift_tab,
                      m1_tab, m2_tab])
    X = jnp.concatenate(
        [w.astype(f32), a.astype(f32), s.astype(f32), d.astype(f32),
         space.astype(f32), shift.astype(f32), dx, dy,
         mouse_1.astype(f32), mouse_2.astype(f32)], axis=1)
    X2000 = X
    XD = X.reshape(B, NROW, GROUP)[:, 6 * NG:8 * NG, :]
    lnw2 = ln_w.reshape(1, OUT)

    TBL, CD, KX, KY, B1, MX, MY, P = pl.pallas_call(
        _prologue_body,
        out_shape=[
            jax.ShapeDtypeStruct((TPAD, OUT), f32),
            jax.ShapeDtypeStruct((2 * NG, OUT), f32),
            jax.ShapeDtypeStruct((GROUP, OUT), f32),
            jax.ShapeDtypeStruct((GROUP, OUT), f32),
            jax.ShapeDtypeStruct((2, OUT), f32),
            jax.ShapeDtypeStruct((OUT, OUT), f32),
            jax.ShapeDtypeStruct((OUT, OUT), f32),
            jax.ShapeDtypeStruct((GROUP * NROW, NROW), f32),
        ],
    )(tabs, dx_W1, dy_W1, dx_b1.reshape(1, HID), dy_b1.reshape(1, HID),
      dx_W2, dy_W2, dx_b2.reshape(1, HID), dy_b2.reshape(1, HID),
      ffn_W, ffn_b.reshape(1, OUT), lnw2)

    Bt = 32
    full = lambda shape: pl.BlockSpec(shape, lambda i: (0,) * len(shape))
    idx, DXY = pl.pallas_call(
        _prep_body,
        grid=(B // Bt,),
        in_specs=[
            pl.BlockSpec((Bt, GROUP * NROW), lambda i: (i, 0)),
            pl.BlockSpec((Bt, 2 * NG, GROUP), lambda i: (i, 0, 0)),
            full((GROUP * NROW, NROW)),
            full((2 * NG, OUT)), full((GROUP, OUT)), full((GROUP, OUT)),
            full((2, OUT)), full((OUT, OUT)), full((OUT, OUT)),
            full((1, OUT)),
        ],
        out_specs=[
            pl.BlockSpec((Bt, 8, 128), lambda i: (i, 0, 0)),
            pl.BlockSpec((Bt * 2 * NG, OUT), lambda i: (i, 0)),
        ],
        out_shape=[
            jax.ShapeDtypeStruct((B, 8, 128), jnp.int32),
            jax.ShapeDtypeStruct((B * 2 * NG, OUT), f32),
        ],
    )(X2000, XD, P, CD, KX, KY, B1, MX, MY, lnw2)

    SRCALL = jnp.concatenate([TBL, DXY], axis=0)

    info = plsc.get_sparse_core_info()
    nsub = info.num_cores * info.num_subcores
    out = _make_sc_gather(B, nsub)(SRCALL, idx)
    return out


# speedup vs baseline: 1.3317x; 1.0293x over previous
import functools
import math

import jax
import jax.numpy as jnp
from jax import lax
from jax.experimental import pallas as pl
from jax.experimental.pallas import tpu as pltpu
from jax.experimental.pallas import tpu_sc as plsc

HID = 64
GROUP = 4
OUT = 256
NSRC = 10
_TAB_OF_SRC = [0, 1, 2, 3, 4, 5, None, None, 6, 7]
NROW = 500
NG = 50
NNIB = 16
TPAD = 9600


def _prologue_body(tabs_ref, dxW1_ref, dyW1_ref, dxb1_ref, dyb1_ref,
                   dxW2_ref, dyW2_ref, dxb2_ref, dyb2_ref,
                   ffnW_ref, ffnb_ref, lnw_ref,
                   TBL_ref, CD_ref, KX_ref, KY_ref, B1_ref, MX_ref, MY_ref,
                   P_ref):
    W = ffnW_ref[...]
    T0 = tabs_ref[:, 0, :]
    DT = tabs_ref[:, 1, :] - T0
    T0t = jnp.concatenate([T0] * GROUP, axis=1)
    b2x = jnp.concatenate([dxb2_ref[...]] * GROUP, axis=1)
    b2y = jnp.concatenate([dyb2_ref[...]] * GROUP, axis=1)
    SRC = jnp.concatenate([T0t[0:6], b2x, b2y, T0t[6:8]], axis=0)
    BASE10 = jnp.dot(SRC, W, preferred_element_type=jnp.float32)

    row = jax.lax.broadcasted_iota(jnp.int32, (NROW, OUT), 0).astype(jnp.float32)
    col = jax.lax.broadcasted_iota(jnp.int32, (NROW, OUT), 1)
    j = col // HID
    d = col % HID
    p = row * float(GROUP) + j.astype(jnp.float32)
    dd = ((d // 2) * 2).astype(jnp.float32)
    freq = jnp.exp(dd * (-math.log(10000.0) / HID))
    ang = p * freq
    pe_r = jnp.where(d % 2 == 0, jnp.sin(ang), jnp.cos(ang))

    C = jnp.dot(pe_r, W, preferred_element_type=jnp.float32) + ffnb_ref[...]
    C = C + jnp.broadcast_to(BASE10[:, None, :], (NSRC, NG, OUT)
                             ).reshape(NROW, OUT)
    CD_ref[...] = C[6 * NG:8 * NG]

    DJ = []
    for jj in range(GROUP):
        Wj = W[HID * jj:HID * (jj + 1), :]
        DJ.append(jnp.dot(DT, Wj, preferred_element_type=jnp.float32))

    ni = jax.lax.broadcasted_iota(jnp.int32, (NNIB, GROUP), 0)
    ji = jax.lax.broadcasted_iota(jnp.int32, (NNIB, GROUP), 1)
    NB = ((ni >> ji) & 1).astype(jnp.float32)

    lnw = lnw_ref[...]
    for s10 in range(NSRC):
        m = _TAB_OF_SRC[s10]
        if m is None:
            contrib = jnp.zeros((NNIB, OUT), jnp.float32)
        else:
            Ds = jnp.concatenate([DJ[jj][m:m + 1] for jj in range(GROUP)],
                                 axis=0)
            contrib = jnp.dot(NB, Ds, preferred_element_type=jnp.float32)
        pre = C[NG * s10:NG * (s10 + 1)][:, None, :] + contrib[None]
        ms = jnp.mean(pre * pre, axis=-1, keepdims=True)
        nrm = (pre * jax.lax.rsqrt(ms + 1e-6) * lnw[None]).reshape(
            NG * NNIB, OUT)
        TBL_ref[pl.ds(s10 * NG * NNIB, NG * NNIB), :] = nrm
    TBL_ref[pl.ds(NROW * NNIB, TPAD - NROW * NNIB), :] = jnp.zeros(
        (TPAD - NROW * NNIB, OUT), jnp.float32)

    zero = jnp.zeros((1, HID), jnp.float32)
    rowsx, rowsy = [], []
    for jj in range(GROUP):
        px = [dxW1_ref[...] if k == jj else zero for k in range(GROUP)]
        py = [dyW1_ref[...] if k == jj else zero for k in range(GROUP)]
        rowsx.append(jnp.concatenate(px, axis=1))
        rowsy.append(jnp.concatenate(py, axis=1))
    KX_ref[...] = jnp.concatenate(rowsx, axis=0)
    KY_ref[...] = jnp.concatenate(rowsy, axis=0)
    B1_ref[...] = jnp.concatenate(
        [jnp.concatenate([dxb1_ref[...]] * GROUP, axis=1),
         jnp.concatenate([dyb1_ref[...]] * GROUP, axis=1)], axis=0)

    mx, my = [], []
    for jj in range(GROUP):
        Wj = W[HID * jj:HID * (jj + 1), :]
        mx.append(jnp.dot(dxW2_ref[...], Wj, preferred_element_type=jnp.float32))
        my.append(jnp.dot(dyW2_ref[...], Wj, preferred_element_type=jnp.float32))
    MX_ref[...] = jnp.concatenate(mx, axis=0)
    MY_ref[...] = jnp.concatenate(my, axis=0)

    ti = jax.lax.broadcasted_iota(jnp.int32, (GROUP * NROW, NROW), 0)
    gi2 = jax.lax.broadcasted_iota(jnp.int32, (GROUP * NROW, NROW), 1)
    P_ref[...] = jnp.where(ti // GROUP == gi2,
                           (1 << (ti % GROUP)), 0).astype(jnp.float32)


_CHUNKS = [(0, 96), (96, 96), (192, 96), (288, 96), (384, 96), (480, 16)]
_NTAIL = 4


def _prep_body(X2_ref, XD_ref, P_ref, CD_ref, KX_ref, KY_ref, B1_ref,
               MX_ref, MY_ref, lnw_ref, tblpad_ref, idx_ref, DXY_ref):
    del tblpad_ref
    Bt = X2_ref.shape[0]
    lnw = lnw_ref[...]

    nib = jnp.dot(X2_ref[...], P_ref[...],
                  preferred_element_type=jnp.float32)
    g = jax.lax.broadcasted_iota(jnp.int32, (1, NROW), 1).astype(jnp.float32)
    bcol = (pl.program_id(0) * Bt
            + jax.lax.broadcasted_iota(jnp.int32, (Bt, 1), 0)).astype(
                jnp.float32)
    gi = jax.lax.broadcasted_iota(jnp.int32, (1, NROW), 1)
    dxyrow = float(TPAD) + bcol * float(2 * NG) + (
        g - float(6 * NG))
    in_dxy = jnp.logical_and(gi >= 6 * NG, gi < 8 * NG)
    idx = jnp.where(in_dxy, dxyrow, nib + float(NNIB) * g).astype(jnp.int32)

    last = idx[:, NROW - 1:NROW]
    idxp = jnp.concatenate([idx] + [last] * (512 - NROW), axis=1)
    rows = []
    for o, n in _CHUNKS:
        take = min(128, 512 - o)
        r = idxp[:, o:o + take]
        if take < 128:
            r = jnp.concatenate([r, jnp.zeros((Bt, 128 - take), jnp.int32)],
                                axis=1)
        rows.append(r)
    tail = jnp.concatenate(
        [idxp[:, NROW - _NTAIL:NROW],
         jnp.zeros((Bt, 124), jnp.int32)], axis=1)
    rows.append(tail)
    rows.append(jnp.zeros((Bt, 128), jnp.int32))
    idx_ref[...] = jnp.stack(rows, axis=1)

    XD = XD_ref[...]
    parts = []
    for half, (K1_ref, M_ref, ib) in enumerate(((KX_ref, MX_ref, 0),
                                                (KY_ref, MY_ref, 1))):
        feats = XD[:, half * NG:(half + 1) * NG, :].reshape(Bt * NG, GROUP)
        pre = jnp.dot(feats, K1_ref[...], preferred_element_type=jnp.float32)
        h = jnp.maximum(pre + B1_ref[ib][None], 0.0)
        parts.append(jnp.dot(h, M_ref[...], preferred_element_type=jnp.float32)
                     .reshape(Bt, NG, OUT))
    acc = jnp.concatenate(parts, axis=1) + CD_ref[...][None]
    ms = jnp.mean(acc * acc, axis=-1, keepdims=True)
    DXY_ref[...] = (acc * jax.lax.rsqrt(ms + 1e-6) * lnw[None]).reshape(
        Bt * 2 * NG, OUT)


def _make_sc_gather(B, nsub):
    nb = B // nsub
    mesh = plsc.VectorSubcoreMesh(core_axis_name="c", subcore_axis_name="s")

    @functools.partial(
        pl.kernel, mesh=mesh,
        out_type=jax.ShapeDtypeStruct((B, NROW, OUT), jnp.float32),
        scratch_types=[
            pltpu.VMEM((8, 128), jnp.int32),
            pltpu.VMEM((NROW, OUT), jnp.float32),
            pltpu.VMEM((4, OUT), jnp.float32),
            pltpu.SemaphoreType.DMA,
        ],
    )
    def sc_gather(srcall, idxh, out, idx_v, slab, tbuf, semg):
        wid = lax.axis_index("s") * 2 + lax.axis_index("c")

        def body(i, carry):
            b = wid * nb + i
            pltpu.sync_copy(idxh.at[b], idx_v)
            gets = [
                pltpu.async_copy(srcall.at[idx_v.at[c, pl.ds(0, n)]],
                                 slab.at[pl.ds(o, n)], semg)
                for c, (o, n) in enumerate(_CHUNKS)
            ]
            gets.append(
                pltpu.async_copy(srcall.at[idx_v.at[6, pl.ds(0, 4)]],
                                 tbuf, semg))
            for dd in gets:
                dd.wait()
            for r in range(_NTAIL):
                for k in range(OUT // 16):
                    slab[NROW - _NTAIL + r, pl.ds(16 * k, 16)] = (
                        tbuf[r, pl.ds(16 * k, 16)])
            pltpu.sync_copy(slab, out.at[b])
            return carry

        lax.fori_loop(0, nb, body, 0)

    return sc_gather


def kernel(w, a, s, d, space, shift, mouse_1, mouse_2, dx, dy, w_tab, a_tab,
           s_tab, d_tab, space_tab, shift_tab, m1_tab, m2_tab, dx_W1, dx_b1,
           dx_W2, dx_b2, dy_W1, dy_b1, dy_W2, dy_b2, ffn_W, ffn_b, ln_w):
    B, T = w.shape
    f32 = jnp.float32

    tabs = jnp.stack([w_tab, a_tab, s_tab, d_tab, space_tab, shift_tab,
                      m1_tab, m2_tab])
    X = jnp.concatenate(
        [w.astype(f32), a.astype(f32), s.astype(f32), d.astype(f32),
         space.astype(f32), shift.astype(f32), dx, dy,
         mouse_1.astype(f32), mouse_2.astype(f32)], axis=1)
    X2000 = X
    XD = X.reshape(B, NROW, GROUP)[:, 6 * NG:8 * NG, :]
    lnw2 = ln_w.reshape(1, OUT)

    TBL, CD, KX, KY, B1, MX, MY, P = pl.pallas_call(
        _prologue_body,
        out_shape=[
            jax.ShapeDtypeStruct((TPAD, OUT), f32),
            jax.ShapeDtypeStruct((2 * NG, OUT), f32),
            jax.ShapeDtypeStruct((GROUP, OUT), f32),
            jax.ShapeDtypeStruct((GROUP, OUT), f32),
            jax.ShapeDtypeStruct((2, OUT), f32),
            jax.ShapeDtypeStruct((OUT, OUT), f32),
            jax.ShapeDtypeStruct((OUT, OUT), f32),
            jax.ShapeDtypeStruct((GROUP * NROW, NROW), f32),
        ],
    )(tabs, dx_W1, dy_W1, dx_b1.reshape(1, HID), dy_b1.reshape(1, HID),
      dx_W2, dy_W2, dx_b2.reshape(1, HID), dy_b2.reshape(1, HID),
      ffn_W, ffn_b.reshape(1, OUT), lnw2)

    Bt = 32
    full = lambda shape: pl.BlockSpec(shape, lambda i: (0,) * len(shape))
    TBLPAD = jnp.concatenate(
        [TBL, jnp.zeros((B * 2 * NG, OUT), f32)], axis=0)
    idx, SRCALL = pl.pallas_call(
        _prep_body,
        grid=(B // Bt,),
        in_specs=[
            pl.BlockSpec((Bt, GROUP * NROW), lambda i: (i, 0)),
            pl.BlockSpec((Bt, 2 * NG, GROUP), lambda i: (i, 0, 0)),
            full((GROUP * NROW, NROW)),
            full((2 * NG, OUT)), full((GROUP, OUT)), full((GROUP, OUT)),
            full((2, OUT)), full((OUT, OUT)), full((OUT, OUT)),
            full((1, OUT)),
            pl.BlockSpec(memory_space=pltpu.MemorySpace.HBM),
        ],
        out_specs=[
            pl.BlockSpec((Bt, 8, 128), lambda i: (i, 0, 0)),
            pl.BlockSpec((Bt * 2 * NG, OUT),
                         lambda i: (TPAD // (Bt * 2 * NG) + i, 0)),
        ],
        out_shape=[
            jax.ShapeDtypeStruct((B, 8, 128), jnp.int32),
            jax.ShapeDtypeStruct((TPAD + B * 2 * NG, OUT), f32),
        ],
        input_output_aliases={10: 1},
    )(X2000, XD, P, CD, KX, KY, B1, MX, MY, lnw2, TBLPAD)

    info = plsc.get_sparse_core_info()
    nsub = info.num_cores * info.num_subcores
    out = _make_sc_gather(B, nsub)(SRCALL, idx)
    return out


# speedup vs baseline: 1.3384x; 1.0050x over previous
import functools
import math

import jax
import jax.numpy as jnp
from jax import lax
from jax.experimental import pallas as pl
from jax.experimental.pallas import tpu as pltpu
from jax.experimental.pallas import tpu_sc as plsc

HID = 64
GROUP = 4
OUT = 256
NSRC = 10
_TAB_OF_SRC = [0, 1, 2, 3, 4, 5, None, None, 6, 7]
NROW = 500
NG = 50
NNIB = 16
TPAD = 9600


def _prologue_body(tabs_ref, dxW1_ref, dyW1_ref, dxb1_ref, dyb1_ref,
                   dxW2_ref, dyW2_ref, dxb2_ref, dyb2_ref,
                   ffnW_ref, ffnb_ref, lnw_ref,
                   TBL_ref, CD_ref, KX_ref, KY_ref, B1_ref, MX_ref, MY_ref,
                   P_ref):
    W = ffnW_ref[...]
    T0 = tabs_ref[:, 0, :]
    DT = tabs_ref[:, 1, :] - T0
    T0t = jnp.concatenate([T0] * GROUP, axis=1)
    b2x = jnp.concatenate([dxb2_ref[...]] * GROUP, axis=1)
    b2y = jnp.concatenate([dyb2_ref[...]] * GROUP, axis=1)
    SRC = jnp.concatenate([T0t[0:6], b2x, b2y, T0t[6:8]], axis=0)
    BASE10 = jnp.dot(SRC, W, preferred_element_type=jnp.float32)

    row = jax.lax.broadcasted_iota(jnp.int32, (NROW, OUT), 0).astype(jnp.float32)
    col = jax.lax.broadcasted_iota(jnp.int32, (NROW, OUT), 1)
    j = col // HID
    d = col % HID
    p = row * float(GROUP) + j.astype(jnp.float32)
    dd = ((d // 2) * 2).astype(jnp.float32)
    freq = jnp.exp(dd * (-math.log(10000.0) / HID))
    ang = p * freq
    pe_r = jnp.where(d % 2 == 0, jnp.sin(ang), jnp.cos(ang))

    C = jnp.dot(pe_r, W, preferred_element_type=jnp.float32) + ffnb_ref[...]
    C = C + jnp.broadcast_to(BASE10[:, None, :], (NSRC, NG, OUT)
                             ).reshape(NROW, OUT)
    CD_ref[...] = C[6 * NG:8 * NG]

    DJ = []
    for jj in range(GROUP):
        Wj = W[HID * jj:HID * (jj + 1), :]
        DJ.append(jnp.dot(DT, Wj, preferred_element_type=jnp.float32))

    ni = jax.lax.broadcasted_iota(jnp.int32, (NNIB, GROUP), 0)
    ji = jax.lax.broadcasted_iota(jnp.int32, (NNIB, GROUP), 1)
    NB = ((ni >> ji) & 1).astype(jnp.float32)

    lnw = lnw_ref[...]
    for s10 in range(NSRC):
        m = _TAB_OF_SRC[s10]
        if m is None:
            contrib = jnp.zeros((NNIB, OUT), jnp.float32)
        else:
            Ds = jnp.concatenate([DJ[jj][m:m + 1] for jj in range(GROUP)],
                                 axis=0)
            contrib = jnp.dot(NB, Ds, preferred_element_type=jnp.float32)
        pre = C[NG * s10:NG * (s10 + 1)][:, None, :] + contrib[None]
        ms = jnp.mean(pre * pre, axis=-1, keepdims=True)
        nrm = (pre * jax.lax.rsqrt(ms + 1e-6) * lnw[None]).reshape(
            NG * NNIB, OUT)
        TBL_ref[pl.ds(s10 * NG * NNIB, NG * NNIB), :] = nrm
    TBL_ref[pl.ds(NROW * NNIB, TPAD - NROW * NNIB), :] = jnp.zeros(
        (TPAD - NROW * NNIB, OUT), jnp.float32)

    zero = jnp.zeros((1, HID), jnp.float32)
    rowsx, rowsy = [], []
    for jj in range(GROUP):
        px = [dxW1_ref[...] if k == jj else zero for k in range(GROUP)]
        py = [dyW1_ref[...] if k == jj else zero for k in range(GROUP)]
        rowsx.append(jnp.concatenate(px, axis=1))
        rowsy.append(jnp.concatenate(py, axis=1))
    KX_ref[...] = jnp.concatenate(rowsx, axis=0)
    KY_ref[...] = jnp.concatenate(rowsy, axis=0)
    B1_ref[...] = jnp.concatenate(
        [jnp.concatenate([dxb1_ref[...]] * GROUP, axis=1),
         jnp.concatenate([dyb1_ref[...]] * GROUP, axis=1)], axis=0)

    mx, my = [], []
    for jj in range(GROUP):
        Wj = W[HID * jj:HID * (jj + 1), :]
        mx.append(jnp.dot(dxW2_ref[...], Wj, preferred_element_type=jnp.float32))
        my.append(jnp.dot(dyW2_ref[...], Wj, preferred_element_type=jnp.float32))
    MX_ref[...] = jnp.concatenate(mx, axis=0)
    MY_ref[...] = jnp.concatenate(my, axis=0)

    ti = jax.lax.broadcasted_iota(jnp.int32, (GROUP * NROW, NROW), 0)
    gi2 = jax.lax.broadcasted_iota(jnp.int32, (GROUP * NROW, NROW), 1)
    P_ref[...] = jnp.where(ti // GROUP == gi2,
                           (1 << (ti % GROUP)), 0).astype(jnp.float32)


_CHUNKS = [(0, 96), (96, 96), (192, 96), (288, 96), (384, 96), (480, 16)]
_NTAIL = 4


def _prep_body(X2_ref, XD_ref, P_ref, CD_ref, KX_ref, KY_ref, B1_ref,
               MX_ref, MY_ref, lnw_ref, tblpad_ref, idx_ref, DXY_ref):
    del tblpad_ref
    Bt = X2_ref.shape[0]
    lnw = lnw_ref[...]

    nib = jnp.dot(X2_ref[...], P_ref[...],
                  preferred_element_type=jnp.float32)
    g = jax.lax.broadcasted_iota(jnp.int32, (1, NROW), 1).astype(jnp.float32)
    bcol = (pl.program_id(0) * Bt
            + jax.lax.broadcasted_iota(jnp.int32, (Bt, 1), 0)).astype(
                jnp.float32)
    gi = jax.lax.broadcasted_iota(jnp.int32, (1, NROW), 1)
    dxyrow = float(TPAD) + bcol * float(2 * NG) + (
        g - float(6 * NG))
    in_dxy = jnp.logical_and(gi >= 6 * NG, gi < 8 * NG)
    idx = jnp.where(in_dxy, dxyrow, nib + float(NNIB) * g).astype(jnp.int32)

    last = idx[:, NROW - 1:NROW]
    idxp = jnp.concatenate([idx] + [last] * (512 - NROW), axis=1)
    rows = []
    for o, n in _CHUNKS:
        take = min(128, 512 - o)
        r = idxp[:, o:o + take]
        if take < 128:
            r = jnp.concatenate([r, jnp.zeros((Bt, 128 - take), jnp.int32)],
                                axis=1)
        rows.append(r)
    tail = jnp.concatenate(
        [idxp[:, NROW - _NTAIL:NROW],
         jnp.zeros((Bt, 124), jnp.int32)], axis=1)
    rows.append(tail)
    rows.append(jnp.zeros((Bt, 128), jnp.int32))
    idx_ref[...] = jnp.stack(rows, axis=1)

    XD = XD_ref[...]
    parts = []
    for half, (K1_ref, M_ref, ib) in enumerate(((KX_ref, MX_ref, 0),
                                                (KY_ref, MY_ref, 1))):
        feats = XD[:, half * NG:(half + 1) * NG, :].reshape(Bt * NG, GROUP)
        pre = jnp.dot(feats, K1_ref[...], preferred_element_type=jnp.float32)
        h = jnp.maximum(pre + B1_ref[ib][None], 0.0)
        parts.append(jnp.dot(h, M_ref[...], preferred_element_type=jnp.float32)
                     .reshape(Bt, NG, OUT))
    acc = jnp.concatenate(parts, axis=1) + CD_ref[...][None]
    ms = jnp.mean(acc * acc, axis=-1, keepdims=True)
    DXY_ref[...] = (acc * jax.lax.rsqrt(ms + 1e-6) * lnw[None]).reshape(
        Bt * 2 * NG, OUT)


def _make_sc_gather(B, nsub):
    nb = B // nsub
    mesh = plsc.VectorSubcoreMesh(core_axis_name="c", subcore_axis_name="s")

    @functools.partial(
        pl.kernel, mesh=mesh,
        out_type=jax.ShapeDtypeStruct((B, NROW, OUT), jnp.float32),
        scratch_types=[
            pltpu.VMEM((8, 128), jnp.int32),
            pltpu.VMEM((NROW, OUT), jnp.float32),
            pltpu.VMEM((4, OUT), jnp.float32),
            pltpu.SemaphoreType.DMA,
        ],
    )
    def sc_gather(srcall, idxh, out, idx_v, slab, tbuf, semg):
        wid = lax.axis_index("s") * 2 + lax.axis_index("c")

        def body(i, carry):
            b = wid * nb + i
            pltpu.sync_copy(idxh.at[b], idx_v)
            gets = [
                pltpu.async_copy(srcall.at[idx_v.at[c, pl.ds(0, n)]],
                                 slab.at[pl.ds(o, n)], semg)
                for c, (o, n) in enumerate(_CHUNKS)
            ]
            gets.append(
                pltpu.async_copy(srcall.at[idx_v.at[6, pl.ds(0, 4)]],
                                 tbuf, semg))
            for dd in gets:
                dd.wait()
            for r in range(_NTAIL):
                for k in range(OUT // 16):
                    slab[NROW - _NTAIL + r, pl.ds(16 * k, 16)] = (
                        tbuf[r, pl.ds(16 * k, 16)])
            pltpu.sync_copy(slab, out.at[b])
            return carry

        lax.fori_loop(0, nb, body, 0)

    return sc_gather


def kernel(w, a, s, d, space, shift, mouse_1, mouse_2, dx, dy, w_tab, a_tab,
           s_tab, d_tab, space_tab, shift_tab, m1_tab, m2_tab, dx_W1, dx_b1,
           dx_W2, dx_b2, dy_W1, dy_b1, dy_W2, dy_b2, ffn_W, ffn_b, ln_w):
    B, T = w.shape
    f32 = jnp.float32

    tabs = jnp.stack([w_tab, a_tab, s_tab, d_tab, space_tab, shift_tab,
                      m1_tab, m2_tab])
    X = jnp.concatenate(
        [w.astype(f32), a.astype(f32), s.astype(f32), d.astype(f32),
         space.astype(f32), shift.astype(f32), dx, dy,
         mouse_1.astype(f32), mouse_2.astype(f32)], axis=1)
    X2000 = X
    XD = X.reshape(B, NROW, GROUP)[:, 6 * NG:8 * NG, :]
    lnw2 = ln_w.reshape(1, OUT)

    TBL, CD, KX, KY, B1, MX, MY, P = pl.pallas_call(
        _prologue_body,
        out_shape=[
            jax.ShapeDtypeStruct((TPAD, OUT), f32),
            jax.ShapeDtypeStruct((2 * NG, OUT), f32),
            jax.ShapeDtypeStruct((GROUP, OUT), f32),
            jax.ShapeDtypeStruct((GROUP, OUT), f32),
            jax.ShapeDtypeStruct((2, OUT), f32),
            jax.ShapeDtypeStruct((OUT, OUT), f32),
            jax.ShapeDtypeStruct((OUT, OUT), f32),
            jax.ShapeDtypeStruct((GROUP * NROW, NROW), f32),
        ],
    )(tabs, dx_W1, dy_W1, dx_b1.reshape(1, HID), dy_b1.reshape(1, HID),
      dx_W2, dy_W2, dx_b2.reshape(1, HID), dy_b2.reshape(1, HID),
      ffn_W, ffn_b.reshape(1, OUT), lnw2)

    Bt = 64
    full = lambda shape: pl.BlockSpec(shape, lambda i: (0,) * len(shape))
    TBLPAD = jnp.concatenate(
        [TBL, jnp.zeros((B * 2 * NG, OUT), f32)], axis=0)
    idx, SRCALL = pl.pallas_call(
        _prep_body,
        grid=(B // Bt,),
        in_specs=[
            pl.BlockSpec((Bt, GROUP * NROW), lambda i: (i, 0)),
            pl.BlockSpec((Bt, 2 * NG, GROUP), lambda i: (i, 0, 0)),
            full((GROUP * NROW, NROW)),
            full((2 * NG, OUT)), full((GROUP, OUT)), full((GROUP, OUT)),
            full((2, OUT)), full((OUT, OUT)), full((OUT, OUT)),
            full((1, OUT)),
            pl.BlockSpec(memory_space=pltpu.MemorySpace.HBM),
        ],
        out_specs=[
            pl.BlockSpec((Bt, 8, 128), lambda i: (i, 0, 0)),
            pl.BlockSpec((Bt * 2 * NG, OUT),
                         lambda i: (TPAD // (Bt * 2 * NG) + i, 0)),
        ],
        out_shape=[
            jax.ShapeDtypeStruct((B, 8, 128), jnp.int32),
            jax.ShapeDtypeStruct((TPAD + B * 2 * NG, OUT), f32),
        ],
        input_output_aliases={10: 1},
    )(X2000, XD, P, CD, KX, KY, B1, MX, MY, lnw2, TBLPAD)

    info = plsc.get_sparse_core_info()
    nsub = info.num_cores * info.num_subcores
    out = _make_sc_gather(B, nsub)(SRCALL, idx)
    return out
